# baseline (speedup 1.0000x reference)
import jax
import jax.numpy as jnp
from jax.experimental import pallas as pl
from jax.experimental.pallas import tpu as pltpu


def _gating_outer(x_hbm, mask_hbm, w_ref, probs_hbm, logits_hbm):
    w = w_ref[...]
    nsteps = x_hbm.shape[0] // _BT

    def inner(x_ref, mask_ref, probs_ref, logits_ref):
        logits = jax.lax.dot_general(
            x_ref[...],
            w,
            dimension_numbers=(((1,), (1,)), ((), ())),
            preferred_element_type=jnp.float32,
        )
        m = jnp.max(logits, axis=-1, keepdims=True)
        e = jnp.exp(logits - m)
        probs = e / jnp.sum(e, axis=-1, keepdims=True)
        probs_ref[...] = probs * mask_ref[...]
        logits_ref[...] = logits

    pipeline = pltpu.emit_pipeline(
        inner,
        grid=(nsteps,),
        in_specs=[
            pl.BlockSpec((_BT, x_hbm.shape[1]), lambda i: (i, 0),
                         pipeline_mode=pl.Buffered(buffer_count=_NBUF)),
            pl.BlockSpec((_BT, 1), lambda i: (i, 0)),
        ],
        out_specs=[
            pl.BlockSpec((_BT, probs_hbm.shape[1]), lambda i: (i, 0)),
            pl.BlockSpec((_BT, logits_hbm.shape[1]), lambda i: (i, 0)),
        ],
    )
    pipeline(x_hbm, mask_hbm, probs_hbm, logits_hbm)


_BT = 512
_NBUF = 4


def kernel(inputs, padding_mask, W):
    T, H = inputs.shape
    E = W.shape[0]
    mask2d = padding_mask.reshape(T, 1)
    probs, logits = pl.pallas_call(
        _gating_outer,
        in_specs=[
            pl.BlockSpec(memory_space=pl.ANY),
            pl.BlockSpec(memory_space=pl.ANY),
            pl.BlockSpec(memory_space=pltpu.VMEM),
        ],
        out_specs=[
            pl.BlockSpec(memory_space=pl.ANY),
            pl.BlockSpec(memory_space=pl.ANY),
        ],
        out_shape=[
            jax.ShapeDtypeStruct((T, E), jnp.float32),
            jax.ShapeDtypeStruct((T, E), jnp.float32),
        ],
    )(inputs, mask2d, W)
    return (probs, logits)


# speedup vs baseline: 1.0470x; 1.0470x over previous
import jax
import jax.numpy as jnp
from jax.experimental import pallas as pl
from jax.experimental.pallas import tpu as pltpu


def _gating_outer(x_hbm, mask_hbm, w_ref, probs_hbm, logits_hbm):
    w = w_ref[...]
    nsteps = x_hbm.shape[0] // _BT

    def inner(x_ref, mask_ref, probs_ref, logits_ref):
        logits = jax.lax.dot_general(
            x_ref[...],
            w,
            dimension_numbers=(((1,), (1,)), ((), ())),
            preferred_element_type=jnp.float32,
        )
        m = jnp.max(logits, axis=-1, keepdims=True)
        e = jnp.exp(logits - m)
        probs = e / jnp.sum(e, axis=-1, keepdims=True)
        probs_ref[...] = probs * mask_ref[...]
        logits_ref[...] = logits

    pipeline = pltpu.emit_pipeline(
        inner,
        grid=(nsteps,),
        in_specs=[
            pl.BlockSpec((_BT, x_hbm.shape[1]), lambda i: (i, 0),
                         pipeline_mode=pl.Buffered(buffer_count=_NBUF)),
            pl.BlockSpec((_BT, 1), lambda i: (i, 0)),
        ],
        out_specs=[
            pl.BlockSpec((_BT, probs_hbm.shape[1]), lambda i: (i, 0)),
            pl.BlockSpec((_BT, logits_hbm.shape[1]), lambda i: (i, 0)),
        ],
    )
    pipeline(x_hbm, mask_hbm, probs_hbm, logits_hbm)


_BT = 256
_NBUF = 8


def kernel(inputs, padding_mask, W):
    T, H = inputs.shape
    E = W.shape[0]
    mask2d = padding_mask.reshape(T, 1)
    probs, logits = pl.pallas_call(
        _gating_outer,
        in_specs=[
            pl.BlockSpec(memory_space=pl.ANY),
            pl.BlockSpec(memory_space=pl.ANY),
            pl.BlockSpec(memory_space=pltpu.VMEM),
        ],
        out_specs=[
            pl.BlockSpec(memory_space=pl.ANY),
            pl.BlockSpec(memory_space=pl.ANY),
        ],
        out_shape=[
            jax.ShapeDtypeStruct((T, E), jnp.float32),
            jax.ShapeDtypeStruct((T, E), jnp.float32),
        ],
    )(inputs, mask2d, W)
    return (probs, logits)
